# baseline (speedup 1.0000x reference)
import functools

import jax
import jax.numpy as jnp
import numpy as np
from jax.experimental import pallas as pl
from jax.experimental.pallas import tpu as pltpu

KNN = 32
BN = 128
CHUNK = 2048
BIG = np.float32(3.0e38)


def _mid_eigenvalue_3x3(cxx, cyy, czz, cxy, cxz, cyz):
    q = (cxx + cyy + czz) / 3.0
    dxx = cxx - q
    dyy = cyy - q
    dzz = czz - q
    p1 = cxy * cxy + cxz * cxz + cyz * cyz
    p2 = dxx * dxx + dyy * dyy + dzz * dzz + 2.0 * p1
    p = jnp.sqrt(jnp.maximum(p2, 0.0) / 6.0)
    inv_p = 1.0 / jnp.maximum(p, np.float32(1e-18))
    bxx = dxx * inv_p
    byy = dyy * inv_p
    bzz = dzz * inv_p
    bxy = cxy * inv_p
    bxz = cxz * inv_p
    byz = cyz * inv_p
    det_b = (
        bxx * (byy * bzz - byz * byz)
        - bxy * (bxy * bzz - byz * bxz)
        + bxz * (bxy * byz - byy * bxz)
    )
    r = jnp.clip(det_b * 0.5, -1.0, 1.0)
    c = jnp.ones_like(r)
    for _ in range(12):
        g = (4.0 * c * c - 3.0) * c - r
        gp = jnp.maximum(12.0 * c * c - 3.0, np.float32(1e-12))
        c = c - g / gp
    s = jnp.sqrt(jnp.maximum(1.0 - c * c, 0.0))
    return q + p * (np.float32(1.7320508075688772) * s - c)


def _skel_kernel(n_chunks, c_ref, xt_ref, nc_ref, rad_ref, d2_ref):
    c = c_ref[0]
    ionc = jax.lax.broadcasted_iota(jnp.int32, (BN, n_chunks), 1)
    iota_b = jax.lax.broadcasted_iota(jnp.int32, (BN, CHUNK), 1)

    def init_body(ci, carry):
        minc, argc = carry
        xtc = xt_ref[0, ci]
        ex = c[:, 0:1] - xtc[0:1, :]
        ey = c[:, 1:2] - xtc[1:2, :]
        ez = c[:, 2:3] - xtc[2:3, :]
        d2c = ex * ex + ey * ey + ez * ez
        d2_ref[ci] = d2c
        mc = jnp.min(d2c, axis=1, keepdims=True)
        ac = jnp.min(jnp.where(d2c <= mc, iota_b, np.int32(2 ** 30)),
                     axis=1, keepdims=True) + ci * CHUNK
        minc = jnp.where(ionc == ci, mc, minc)
        argc = jnp.where(ionc == ci, ac, argc)
        return minc, argc

    minc = (ionc * 0).astype(jnp.float32) + BIG
    argc = ionc * 0
    minc, argc = jax.lax.fori_loop(0, n_chunks, init_body, (minc, argc))

    dists = []
    nbrx_l = []
    nbry_l = []
    nbrz_l = []
    for k in range(KNN):
        m = jnp.min(minc, axis=1, keepdims=True)
        amin = jnp.min(jnp.where(minc <= m, argc, np.int32(2 ** 30)),
                       axis=1, keepdims=True)
        dists.append(jnp.sqrt(jnp.maximum(m, 0.0)))
        last = k == KNN - 1

        def ext_body(ci, carry, amin=amin, m=m, last=last):
            minc, argc, nx, ny, nz = carry
            d2c = d2_ref[ci]
            aminl = amin - ci * CHUNK
            eq = iota_b == aminl
            xtc = xt_ref[0, ci]
            zero = np.float32(0.0)
            nx = nx + jnp.sum(jnp.where(eq, xtc[0:1, :], zero),
                              axis=1, keepdims=True)
            ny = ny + jnp.sum(jnp.where(eq, xtc[1:2, :], zero),
                              axis=1, keepdims=True)
            nz = nz + jnp.sum(jnp.where(eq, xtc[2:3, :], zero),
                              axis=1, keepdims=True)
            if not last:
                invalid = (d2c < m) | ((d2c == m) & (iota_b <= aminl))
                dm = jnp.where(invalid, BIG, d2c)
                mc = jnp.min(dm, axis=1, keepdims=True)
                ac = jnp.min(jnp.where(dm <= mc, iota_b, np.int32(2 ** 30)),
                             axis=1, keepdims=True) + ci * CHUNK
                minc = jnp.where(ionc == ci, mc, minc)
                argc = jnp.where(ionc == ci, ac, argc)
            return minc, argc, nx, ny, nz

        z = (jax.lax.broadcasted_iota(jnp.int32, (BN, 1), 1) * 0
             ).astype(jnp.float32)
        minc, argc, nx, ny, nz = jax.lax.fori_loop(
            0, n_chunks, ext_body, (minc, argc, z, z, z))
        nbrx_l.append(nx)
        nbry_l.append(ny)
        nbrz_l.append(nz)

    dist = jnp.concatenate(dists, axis=1)
    nbrx = jnp.concatenate(nbrx_l, axis=1)
    nbry = jnp.concatenate(nbry_l, axis=1)
    nbrz = jnp.concatenate(nbrz_l, axis=1)

    w1 = np.float32(1.0) / (np.float32(KNN) + np.float32(1e-7))
    mx = jnp.sum(nbrx, axis=1, keepdims=True) * w1
    my = jnp.sum(nbry, axis=1, keepdims=True) * w1
    mz = jnp.sum(nbrz, axis=1, keepdims=True) * w1
    dx = nbrx - mx
    dy = nbry - my
    dz = nbrz - mz
    cxx = jnp.sum(dx * dx, axis=1, keepdims=True) * w1
    cyy = jnp.sum(dy * dy, axis=1, keepdims=True) * w1
    czz = jnp.sum(dz * dz, axis=1, keepdims=True) * w1
    cxy = jnp.sum(dx * dy, axis=1, keepdims=True) * w1
    cxz = jnp.sum(dx * dz, axis=1, keepdims=True) * w1
    cyz = jnp.sum(dy * dz, axis=1, keepdims=True) * w1

    lam_mid = _mid_eigenvalue_3x3(cxx, cyy, czz, cxy, cxz, cyz)
    radius = jnp.sqrt(jnp.maximum(lam_mid, 0.0))

    mask = (dist - 3.0 * radius <= 0.0).astype(jnp.float32)
    denom = jnp.sum(mask, axis=1, keepdims=True) + np.float32(1e-7)
    ncx = jnp.sum(nbrx * mask, axis=1, keepdims=True) / denom
    ncy = jnp.sum(nbry * mask, axis=1, keepdims=True) / denom
    ncz = jnp.sum(nbrz * mask, axis=1, keepdims=True) / denom

    nc_ref[0] = jnp.concatenate([ncx, ncy, ncz], axis=1)
    rad_ref[0] = radius


def _build_call(B, N, M):
    n_chunks = M // CHUNK
    kfn = functools.partial(_skel_kernel, n_chunks)
    return pl.pallas_call(
        kfn,
        grid=(B, N // BN),
        in_specs=[
            pl.BlockSpec((1, BN, 3), lambda b, i: (b, i, 0)),
            pl.BlockSpec((1, n_chunks, 3, CHUNK), lambda b, i: (b, 0, 0, 0)),
        ],
        out_specs=[
            pl.BlockSpec((1, BN, 3), lambda b, i: (b, i, 0)),
            pl.BlockSpec((1, BN, 1), lambda b, i: (b, i, 0)),
        ],
        out_shape=[
            jax.ShapeDtypeStruct((B, N, 3), jnp.float32),
            jax.ShapeDtypeStruct((B, N, 1), jnp.float32),
        ],
        scratch_shapes=[pltpu.VMEM((n_chunks, BN, CHUNK), jnp.float32)],
        compiler_params=pltpu.CompilerParams(
            dimension_semantics=("parallel", "parallel"),
        ),
    )


def kernel(centers, xyz):
    B, N, _ = centers.shape
    M = xyz.shape[1]
    n_chunks = M // CHUNK
    xt = jnp.transpose(xyz, (0, 2, 1)).reshape(B, 3, n_chunks, CHUNK)
    xt = jnp.transpose(xt, (0, 2, 1, 3))
    new_centers, radius = _build_call(B, N, M)(centers, xt)
    return (new_centers, radius)


# speedup vs baseline: 1.5628x; 1.5628x over previous
import functools

import jax
import jax.numpy as jnp
import numpy as np
from jax.experimental import pallas as pl
from jax.experimental.pallas import tpu as pltpu

KNN = 32
BN = 128
CHUNK = 2048
BIG = np.float32(3.0e38)


def _mid_eigenvalue_3x3(cxx, cyy, czz, cxy, cxz, cyz):
    q = (cxx + cyy + czz) / 3.0
    dxx = cxx - q
    dyy = cyy - q
    dzz = czz - q
    p1 = cxy * cxy + cxz * cxz + cyz * cyz
    p2 = dxx * dxx + dyy * dyy + dzz * dzz + 2.0 * p1
    p = jnp.sqrt(jnp.maximum(p2, 0.0) / 6.0)
    inv_p = 1.0 / jnp.maximum(p, np.float32(1e-18))
    bxx = dxx * inv_p
    byy = dyy * inv_p
    bzz = dzz * inv_p
    bxy = cxy * inv_p
    bxz = cxz * inv_p
    byz = cyz * inv_p
    det_b = (
        bxx * (byy * bzz - byz * byz)
        - bxy * (bxy * bzz - byz * bxz)
        + bxz * (bxy * byz - byy * bxz)
    )
    r = jnp.clip(det_b * 0.5, -1.0, 1.0)
    c = jnp.ones_like(r)
    for _ in range(12):
        g = (4.0 * c * c - 3.0) * c - r
        gp = jnp.maximum(12.0 * c * c - 3.0, np.float32(1e-12))
        c = c - g / gp
    s = jnp.sqrt(jnp.maximum(1.0 - c * c, 0.0))
    return q + p * (np.float32(1.7320508075688772) * s - c)


def _skel_kernel(n_chunks, c_ref, xt_ref, nc_ref, rad_ref, d2_ref, d2p_ref):
    c = c_ref[0]
    ionc = jax.lax.broadcasted_iota(jnp.int32, (BN, n_chunks), 1)
    iota_b = jax.lax.broadcasted_iota(jnp.int32, (BN, CHUNK), 1)

    def init_body(ci, carry):
        minc, argc = carry
        xtc = xt_ref[0, ci]
        ex = c[:, 0:1] - xtc[0:1, :]
        ey = c[:, 1:2] - xtc[1:2, :]
        ez = c[:, 2:3] - xtc[2:3, :]
        d2c = ex * ex + ey * ey + ez * ez
        d2_ref[ci] = d2c
        d2p_ref[ci] = d2c
        mc = jnp.min(d2c, axis=1, keepdims=True)
        ac = jnp.min(jnp.where(d2c <= mc, iota_b, np.int32(2 ** 30)),
                     axis=1, keepdims=True) + ci * CHUNK
        minc = jnp.where(ionc == ci, mc, minc)
        argc = jnp.where(ionc == ci, ac, argc)
        return minc, argc

    minc = (ionc * 0).astype(jnp.float32) + BIG
    argc = ionc * 0
    minc, argc = jax.lax.fori_loop(0, n_chunks, init_body, (minc, argc))

    m = amin = None
    for k in range(KNN):
        m = jnp.min(minc, axis=1, keepdims=True)
        amin = jnp.min(jnp.where(minc <= m, argc, np.int32(2 ** 30)),
                       axis=1, keepdims=True)
        if k == KNN - 1:
            break

        def ext_body(ci, carry, amin=amin):
            minc, argc = carry
            d2c = d2_ref[ci]
            aminl = amin - ci * CHUNK
            eq = iota_b == aminl
            d2c = jnp.where(eq, BIG, d2c)
            d2_ref[ci] = d2c
            mc = jnp.min(d2c, axis=1, keepdims=True)
            ac = jnp.min(jnp.where(d2c <= mc, iota_b, np.int32(2 ** 30)),
                         axis=1, keepdims=True) + ci * CHUNK
            minc = jnp.where(ionc == ci, mc, minc)
            argc = jnp.where(ionc == ci, ac, argc)
            return minc, argc

        minc, argc = jax.lax.fori_loop(0, n_chunks, ext_body, (minc, argc))

    zero = np.float32(0.0)

    def sel_mask(ci):
        aminl = amin - ci * CHUNK
        return (d2_ref[ci] == BIG) | (iota_b == aminl)

    def sum_body(ci, carry):
        sx, sy, sz = carry
        xtc = xt_ref[0, ci]
        sel = sel_mask(ci)
        sx = sx + jnp.sum(jnp.where(sel, xtc[0:1, :], zero),
                          axis=1, keepdims=True)
        sy = sy + jnp.sum(jnp.where(sel, xtc[1:2, :], zero),
                          axis=1, keepdims=True)
        sz = sz + jnp.sum(jnp.where(sel, xtc[2:3, :], zero),
                          axis=1, keepdims=True)
        return sx, sy, sz

    z1 = (jax.lax.broadcasted_iota(jnp.int32, (BN, 1), 1) * 0
          ).astype(jnp.float32)
    sx, sy, sz = jax.lax.fori_loop(0, n_chunks, sum_body, (z1, z1, z1))

    w1 = np.float32(1.0) / (np.float32(KNN) + np.float32(1e-7))
    mx = sx * w1
    my = sy * w1
    mz = sz * w1

    def cov_body(ci, carry):
        cxx, cyy, czz, cxy, cxz, cyz = carry
        xtc = xt_ref[0, ci]
        sel = sel_mask(ci)
        dx = jnp.where(sel, xtc[0:1, :] - mx, zero)
        dy = jnp.where(sel, xtc[1:2, :] - my, zero)
        dz = jnp.where(sel, xtc[2:3, :] - mz, zero)
        cxx = cxx + jnp.sum(dx * dx, axis=1, keepdims=True)
        cyy = cyy + jnp.sum(dy * dy, axis=1, keepdims=True)
        czz = czz + jnp.sum(dz * dz, axis=1, keepdims=True)
        cxy = cxy + jnp.sum(dx * dy, axis=1, keepdims=True)
        cxz = cxz + jnp.sum(dx * dz, axis=1, keepdims=True)
        cyz = cyz + jnp.sum(dy * dz, axis=1, keepdims=True)
        return cxx, cyy, czz, cxy, cxz, cyz

    cxx, cyy, czz, cxy, cxz, cyz = jax.lax.fori_loop(
        0, n_chunks, cov_body, (z1, z1, z1, z1, z1, z1))

    lam_mid = _mid_eigenvalue_3x3(cxx * w1, cyy * w1, czz * w1,
                                  cxy * w1, cxz * w1, cyz * w1)
    radius = jnp.sqrt(jnp.maximum(lam_mid, 0.0))

    def out_body(ci, carry):
        gx, gy, gz, cnt = carry
        xtc = xt_ref[0, ci]
        sel = sel_mask(ci)
        dist = jnp.sqrt(jnp.maximum(d2p_ref[ci], 0.0))
        sel2 = sel & (dist - 3.0 * radius <= 0.0)
        gx = gx + jnp.sum(jnp.where(sel2, xtc[0:1, :], zero),
                          axis=1, keepdims=True)
        gy = gy + jnp.sum(jnp.where(sel2, xtc[1:2, :], zero),
                          axis=1, keepdims=True)
        gz = gz + jnp.sum(jnp.where(sel2, xtc[2:3, :], zero),
                          axis=1, keepdims=True)
        cnt = cnt + jnp.sum(sel2.astype(jnp.float32), axis=1, keepdims=True)
        return gx, gy, gz, cnt

    gx, gy, gz, cnt = jax.lax.fori_loop(
        0, n_chunks, out_body, (z1, z1, z1, z1))
    denom = cnt + np.float32(1e-7)
    ncx = gx / denom
    ncy = gy / denom
    ncz = gz / denom

    nc_ref[0] = jnp.concatenate([ncx, ncy, ncz], axis=1)
    rad_ref[0] = radius


def _build_call(B, N, M):
    n_chunks = M // CHUNK
    kfn = functools.partial(_skel_kernel, n_chunks)
    return pl.pallas_call(
        kfn,
        grid=(B, N // BN),
        in_specs=[
            pl.BlockSpec((1, BN, 3), lambda b, i: (b, i, 0)),
            pl.BlockSpec((1, n_chunks, 3, CHUNK), lambda b, i: (b, 0, 0, 0)),
        ],
        out_specs=[
            pl.BlockSpec((1, BN, 3), lambda b, i: (b, i, 0)),
            pl.BlockSpec((1, BN, 1), lambda b, i: (b, i, 0)),
        ],
        out_shape=[
            jax.ShapeDtypeStruct((B, N, 3), jnp.float32),
            jax.ShapeDtypeStruct((B, N, 1), jnp.float32),
        ],
        scratch_shapes=[pltpu.VMEM((n_chunks, BN, CHUNK), jnp.float32),
                        pltpu.VMEM((n_chunks, BN, CHUNK), jnp.float32)],
        compiler_params=pltpu.CompilerParams(
            dimension_semantics=("parallel", "parallel"),
        ),
    )


def kernel(centers, xyz):
    B, N, _ = centers.shape
    M = xyz.shape[1]
    n_chunks = M // CHUNK
    xt = jnp.transpose(xyz, (0, 2, 1)).reshape(B, 3, n_chunks, CHUNK)
    xt = jnp.transpose(xt, (0, 2, 1, 3))
    new_centers, radius = _build_call(B, N, M)(centers, xt)
    return (new_centers, radius)
